# baseline (speedup 1.0000x reference)
import functools

import jax
import jax.numpy as jnp
from jax import lax
from jax.experimental import pallas as pl
from jax.experimental.pallas import tpu as pltpu
from jax.experimental.pallas import tpu_sc as plsc

WORD_DIM = 64
INPUT_DIM = 128
BATCH = 16384
VOCAB = 1000000

NC = 2
NS = 16
NW = NC * NS
PER = BATCH // NW
GRP = 16

_sc_mesh = plsc.VectorSubcoreMesh(core_axis_name="c", subcore_axis_name="s")


@functools.partial(
    pl.kernel,
    mesh=_sc_mesh,
    out_type=jax.ShapeDtypeStruct((BATCH, 2 * WORD_DIM), jnp.float32),
    scratch_types=[
        pltpu.VMEM((PER,), jnp.int32),
        pltpu.VMEM((GRP, 2 * WORD_DIM), jnp.float32),
        pltpu.SemaphoreType.DMA,
        pltpu.SemaphoreType.DMA,
    ],
)
def _gather_sc(idx_hbm, table2_hbm, out_hbm, my_idx, stage, sem_g, sem_p):
    wid = lax.axis_index("s") * NC + lax.axis_index("c")
    row0 = wid * PER
    pltpu.sync_copy(idx_hbm.at[pl.ds(row0, PER)], my_idx)

    def group_body(g, carry):
        idxv = my_idx[pl.ds(g * GRP, GRP)] >> 1
        pltpu.async_copy(
            table2_hbm.at[plsc.Indices(idxv)], stage, sem_g
        ).wait()
        pltpu.async_copy(
            stage, out_hbm.at[pl.ds(row0 + g * GRP, GRP)], sem_p
        ).wait()
        return carry

    lax.fori_loop(0, PER // GRP, group_body, 0)


_BLK = 2048


def _proj_body(i_ref, g_ref, w_ref, b_ref, o_ref):
    pair = g_ref[...]
    odd = (i_ref[...] & 1) == 1
    x = jnp.where(odd, pair[:, WORD_DIM:], pair[:, :WORD_DIM])
    acc = jnp.dot(x, w_ref[...], preferred_element_type=jnp.float32)
    o_ref[...] = jnp.tanh(acc + b_ref[...])


def _proj_tc(idx, g, W, b):
    return pl.pallas_call(
        _proj_body,
        grid=(BATCH // _BLK,),
        in_specs=[
            pl.BlockSpec((_BLK, 1), lambda i: (i, 0)),
            pl.BlockSpec((_BLK, 2 * WORD_DIM), lambda i: (i, 0)),
            pl.BlockSpec((WORD_DIM, INPUT_DIM), lambda i: (0, 0)),
            pl.BlockSpec((1, INPUT_DIM), lambda i: (0, 0)),
        ],
        out_specs=pl.BlockSpec((_BLK, INPUT_DIM), lambda i: (i, 0)),
        out_shape=jax.ShapeDtypeStruct((BATCH, INPUT_DIM), jnp.float32),
    )(idx, g, W, b.reshape(1, INPUT_DIM))


def kernel(word_indices, word_table, W, b):
    idx = word_indices.astype(jnp.int32)
    table2 = word_table.reshape(VOCAB // 2, 2 * WORD_DIM)
    g = _gather_sc(idx, table2)
    return _proj_tc(idx.reshape(BATCH, 1), g, W, b)


# speedup vs baseline: 1.0249x; 1.0249x over previous
import functools

import jax
import jax.numpy as jnp
from jax import lax
from jax.experimental import pallas as pl
from jax.experimental.pallas import tpu as pltpu
from jax.experimental.pallas import tpu_sc as plsc

WORD_DIM = 64
INPUT_DIM = 128
BATCH = 16384
VOCAB = 1000000

NC = 2
NS = 16
NW = NC * NS
PER = BATCH // NW
GRP = 16

_sc_mesh = plsc.VectorSubcoreMesh(core_axis_name="c", subcore_axis_name="s")


@functools.partial(
    pl.kernel,
    mesh=_sc_mesh,
    out_type=jax.ShapeDtypeStruct((BATCH, 2 * WORD_DIM), jnp.float32),
    scratch_types=[
        pltpu.VMEM((PER,), jnp.int32),
        pltpu.VMEM((GRP, 2 * WORD_DIM), jnp.float32),
        pltpu.VMEM((GRP, 2 * WORD_DIM), jnp.float32),
        pltpu.SemaphoreType.DMA,
        pltpu.SemaphoreType.DMA,
        pltpu.SemaphoreType.DMA,
        pltpu.SemaphoreType.DMA,
    ],
)
def _gather_sc(idx_hbm, table2_hbm, out_hbm, my_idx, stage0, stage1,
               sem_g0, sem_g1, sem_p0, sem_p1):
    wid = lax.axis_index("s") * NC + lax.axis_index("c")
    row0 = wid * PER
    pltpu.sync_copy(idx_hbm.at[pl.ds(row0, PER)], my_idx)

    stages = (stage0, stage1)
    sem_g = (sem_g0, sem_g1)
    sem_p = (sem_p0, sem_p1)
    n = PER // GRP

    def start_gather(g):
        idxv = my_idx[pl.ds(g * GRP, GRP)] >> 1
        return pltpu.async_copy(
            table2_hbm.at[plsc.Indices(idxv)], stages[g % 2], sem_g[g % 2]
        )

    gh = start_gather(0)
    ph_prev = None
    for g in range(n):
        if g + 1 < n:
            if ph_prev is not None:
                ph_prev.wait()
            gh_next = start_gather(g + 1)
        gh.wait()
        ph = pltpu.async_copy(
            stages[g % 2], out_hbm.at[pl.ds(row0 + g * GRP, GRP)],
            sem_p[g % 2],
        )
        if g + 1 < n:
            ph_prev, ph_last = ph, ph
            gh = gh_next
        if g == n - 1:
            ph_prev.wait()
            ph.wait()


_BLK = 2048


def _proj_body(i_ref, g_ref, w_ref, b_ref, o_ref):
    pair = g_ref[...]
    odd = (i_ref[...] & 1) == 1
    x = jnp.where(odd, pair[:, WORD_DIM:], pair[:, :WORD_DIM])
    acc = jnp.dot(x, w_ref[...], preferred_element_type=jnp.float32)
    o_ref[...] = jnp.tanh(acc + b_ref[...])


def _proj_tc(idx, g, W, b):
    return pl.pallas_call(
        _proj_body,
        grid=(BATCH // _BLK,),
        in_specs=[
            pl.BlockSpec((_BLK, 1), lambda i: (i, 0)),
            pl.BlockSpec((_BLK, 2 * WORD_DIM), lambda i: (i, 0)),
            pl.BlockSpec((WORD_DIM, INPUT_DIM), lambda i: (0, 0)),
            pl.BlockSpec((1, INPUT_DIM), lambda i: (0, 0)),
        ],
        out_specs=pl.BlockSpec((_BLK, INPUT_DIM), lambda i: (i, 0)),
        out_shape=jax.ShapeDtypeStruct((BATCH, INPUT_DIM), jnp.float32),
    )(idx, g, W, b.reshape(1, INPUT_DIM))


def kernel(word_indices, word_table, W, b):
    idx = word_indices.astype(jnp.int32)
    table2 = word_table.reshape(VOCAB // 2, 2 * WORD_DIM)
    g = _gather_sc(idx, table2)
    return _proj_tc(idx.reshape(BATCH, 1), g, W, b)


# speedup vs baseline: 1.0284x; 1.0034x over previous
import functools

import jax
import jax.numpy as jnp
from jax import lax
from jax.experimental import pallas as pl
from jax.experimental.pallas import tpu as pltpu
from jax.experimental.pallas import tpu_sc as plsc

WORD_DIM = 64
INPUT_DIM = 128
BATCH = 16384
VOCAB = 1000000

NC = 2
NS = 16
NW = NC * NS
PER = BATCH // NW
GRP = 16

_sc_mesh = plsc.VectorSubcoreMesh(core_axis_name="c", subcore_axis_name="s")


@functools.partial(
    pl.kernel,
    mesh=_sc_mesh,
    out_type=jax.ShapeDtypeStruct((BATCH, 2 * WORD_DIM), jnp.float32),
    scratch_types=(
        [pltpu.VMEM((PER,), jnp.int32)]
        + [pltpu.VMEM((GRP, 2 * WORD_DIM), jnp.float32)] * 4
        + [pltpu.SemaphoreType.DMA] * 8
    ),
)
def _gather_sc(idx_hbm, table2_hbm, out_hbm, my_idx, *bufs):
    stages = bufs[:4]
    sem_g = bufs[4:8]
    sem_p = bufs[8:12]
    NBUF = 4
    wid = lax.axis_index("s") * NC + lax.axis_index("c")
    row0 = wid * PER
    pltpu.sync_copy(idx_hbm.at[pl.ds(row0, PER)], my_idx)

    n = PER // GRP

    def start_gather(g):
        idxv = my_idx[pl.ds(g * GRP, GRP)] >> 1
        return pltpu.async_copy(
            table2_hbm.at[plsc.Indices(idxv)], stages[g % NBUF],
            sem_g[g % NBUF],
        )

    gh = [None] * n
    ph = [None] * n
    for g in range(min(NBUF - 1, n)):
        gh[g] = start_gather(g)
    for g in range(n):
        nxt = g + NBUF - 1
        if nxt < n:
            if ph[nxt - NBUF] is not None:
                ph[nxt - NBUF].wait()
            gh[nxt] = start_gather(nxt)
        gh[g].wait()
        ph[g] = pltpu.async_copy(
            stages[g % NBUF], out_hbm.at[pl.ds(row0 + g * GRP, GRP)],
            sem_p[g % NBUF],
        )
    for g in range(max(0, n - NBUF), n):
        if ph[g] is not None:
            ph[g].wait()


_BLK = 2048


def _proj_body(i_ref, g_ref, w_ref, b_ref, o_ref):
    pair = g_ref[...]
    odd = (i_ref[...] & 1) == 1
    x = jnp.where(odd, pair[:, WORD_DIM:], pair[:, :WORD_DIM])
    acc = jnp.dot(x, w_ref[...], preferred_element_type=jnp.float32)
    o_ref[...] = jnp.tanh(acc + b_ref[...])


def _proj_tc(idx, g, W, b):
    return pl.pallas_call(
        _proj_body,
        grid=(BATCH // _BLK,),
        in_specs=[
            pl.BlockSpec((_BLK, 1), lambda i: (i, 0)),
            pl.BlockSpec((_BLK, 2 * WORD_DIM), lambda i: (i, 0)),
            pl.BlockSpec((WORD_DIM, INPUT_DIM), lambda i: (0, 0)),
            pl.BlockSpec((1, INPUT_DIM), lambda i: (0, 0)),
        ],
        out_specs=pl.BlockSpec((_BLK, INPUT_DIM), lambda i: (i, 0)),
        out_shape=jax.ShapeDtypeStruct((BATCH, INPUT_DIM), jnp.float32),
    )(idx, g, W, b.reshape(1, INPUT_DIM))


def kernel(word_indices, word_table, W, b):
    idx = word_indices.astype(jnp.int32)
    table2 = word_table.reshape(VOCAB // 2, 2 * WORD_DIM)
    g = _gather_sc(idx, table2)
    return _proj_tc(idx.reshape(BATCH, 1), g, W, b)
